# baseline (speedup 1.0000x reference)
import functools

import jax
import jax.numpy as jnp
from jax import lax
from jax.experimental import pallas as pl
from jax.experimental.pallas import tpu as pltpu
from jax.experimental.pallas import tpu_sc as plsc

_N = 10000
_D = 128
_H = 64
_E = 320000

_NC = 2
_NS = 16
_NW = _NC * _NS
_CW = 512
_NCHT = _E // _CW
_CPW = 20
_NCHP = _NW * _CPW
_STG = 24
_NDUMP = 8
_NIT = 10
_RPT = _N // _NIT


def _sc_aggregate(y, zeros_n, src2, dst2):
    mesh = plsc.VectorSubcoreMesh(
        core_axis_name="c", subcore_axis_name="s", num_cores=_NC, num_subcores=_NS
    )
    nacc = _N + _NDUMP

    @functools.partial(
        pl.kernel,
        mesh=mesh,
        out_type=jax.ShapeDtypeStruct((_NC, _N, _H), jnp.float32),
        scratch_types=[
            pltpu.VMEM((_STG, _CW), jnp.int32),
            pltpu.VMEM((_STG, _CW), jnp.int32),
            pltpu.VMEM((2, _CW, _H), jnp.float32),
            pltpu.VMEM_SHARED((nacc, _H), jnp.float32),
            pltpu.SemaphoreType.DMA,
            pltpu.SemaphoreType.DMA,
            pltpu.SemaphoreType.DMA,
            pltpu.SemaphoreType.DMA,
        ],
        compiler_params=pltpu.CompilerParams(use_tc_tiling_on_sc=False),
    )
    def agg(y_hbm, zero_hbm, src_hbm, dst_hbm, out_hbm, src_v, dst_v, rows_v, acc, g0, g1, s0, s1):
        c = lax.axis_index("c")
        s = lax.axis_index("s")
        w = s * _NC + c
        r0 = s * _RPT

        base = w * _CPW
        a0 = base - lax.rem(base, 8)
        off = base - a0
        pltpu.sync_copy(src_hbm.at[pl.ds(a0, _STG)], src_v)
        pltpu.sync_copy(dst_hbm.at[pl.ds(a0, _STG)], dst_v)

        @pl.when(jnp.logical_and(c == 0, s < _NIT))
        def _():
            pltpu.sync_copy(y_hbm.at[pl.ds(r0, _RPT)], acc.at[pl.ds(r0, _RPT)])

        @pl.when(jnp.logical_and(c == 1, s < _NIT))
        def _():
            pltpu.sync_copy(zero_hbm.at[pl.ds(r0, _RPT)], acc.at[pl.ds(r0, _RPT)])

        plsc.subcore_barrier()

        def g_start(j, slot, sem):
            return pltpu.async_copy(y_hbm.at[src_v.at[off + j]], rows_v.at[slot], sem)

        def g_wait(j, slot, sem):
            pltpu.make_async_copy(y_hbm.at[src_v.at[off + j]], rows_v.at[slot], sem).wait()

        def s_start(j, slot, sem):
            return pltpu.async_copy(rows_v.at[slot], acc.at[dst_v.at[off + j]], sem, add=True)

        def s_wait(j, slot, sem):
            pltpu.make_async_copy(rows_v.at[slot], acc.at[dst_v.at[off + j]], sem).wait()

        g_start(0, 0, g0)

        def body(g, carry):
            j0 = 2 * g
            j1 = j0 + 1

            @pl.when(g > 0)
            def _():
                s_wait(j1 - 2, 1, s1)

            g_start(j1, 1, g1)
            g_wait(j0, 0, g0)
            s_start(j0, 0, s0)
            s_wait(j0, 0, s0)

            @pl.when(g < _CPW // 2 - 1)
            def _():
                g_start(j0 + 2, 0, g0)

            g_wait(j1, 1, g1)
            s_start(j1, 1, s1)
            return carry

        lax.fori_loop(0, _CPW // 2, body, 0)
        s_wait(_CPW - 1, 1, s1)

        plsc.subcore_barrier()

        @pl.when(s < _NIT)
        def _():
            pltpu.sync_copy(acc.at[pl.ds(r0, _RPT)], out_hbm.at[c, pl.ds(r0, _RPT)])

    return agg(y, zeros_n, src2, dst2)


def _hdot(h, w):
    a = jnp.dot(h[:, : _H], w, preferred_element_type=jnp.float32)
    b = jnp.dot(h[:, _H :], w, preferred_element_type=jnp.float32)
    return jnp.concatenate([a, b], axis=1)


def _mm_a(x, w1):
    bn = _N // 10

    def body(xa_ref, xb_ref, w_ref, o_ref):
        ya = jnp.dot(xa_ref[...], w_ref[...], preferred_element_type=jnp.float32)
        yb = jnp.dot(xb_ref[...], w_ref[...], preferred_element_type=jnp.float32)
        o_ref[...] = jnp.concatenate([ya, yb], axis=1)

    return pl.pallas_call(
        body,
        grid=(5,),
        in_specs=[
            pl.BlockSpec((bn, _D), lambda i: (i, 0)),
            pl.BlockSpec((bn, _D), lambda i: (i + 5, 0)),
            pl.BlockSpec((_D, _H), lambda i: (0, 0)),
        ],
        out_specs=pl.BlockSpec((bn, 2 * _H), lambda i: (i, 0)),
        out_shape=jax.ShapeDtypeStruct((_N // 2, 2 * _H), jnp.float32),
    )(x, x, w1)


def _mlp_b(p2, b1x, w2, b2x, w3):
    bn = _N // 10

    def body(p_ref, b1_ref, w2_ref, b2_ref, w3_ref, o_ref):
        h = jnp.maximum(p_ref[0] + p_ref[1] + b1_ref[...], 0.0)
        h = jnp.maximum(_hdot(h, w2_ref[...]) + b2_ref[...], 0.0)
        o_ref[...] = _hdot(h, w3_ref[...])

    return pl.pallas_call(
        body,
        grid=(5,),
        in_specs=[
            pl.BlockSpec((2, bn, 2 * _H), lambda i: (0, i, 0)),
            pl.BlockSpec((1, 2 * _H), lambda i: (0, 0)),
            pl.BlockSpec((_H, _H), lambda i: (0, 0)),
            pl.BlockSpec((1, 2 * _H), lambda i: (0, 0)),
            pl.BlockSpec((_H, _H), lambda i: (0, 0)),
        ],
        out_specs=pl.BlockSpec((bn, 2 * _H), lambda i: (i, 0)),
        out_shape=jax.ShapeDtypeStruct((_N // 2, 2 * _H), jnp.float32),
    )(p2, b1x, w2, b2x, w3)


def _mlp_c(q2, b3x, w4, b4r):
    def body(q_ref, b3_ref, w4_ref, b4_ref, o_ref):
        h = jnp.maximum(q_ref[0] + q_ref[1] + b3_ref[...], 0.0)
        o_ref[: _N // 2] = (
            jnp.dot(h[:, : _H], w4_ref[...], preferred_element_type=jnp.float32)
            + b4_ref[...]
        )
        o_ref[_N // 2 :] = (
            jnp.dot(h[:, _H :], w4_ref[...], preferred_element_type=jnp.float32)
            + b4_ref[...]
        )

    return pl.pallas_call(
        body,
        out_shape=jax.ShapeDtypeStruct((_N, _D), jnp.float32),
    )(q2, b3x, w4, b4r)


def kernel(x, edge_index, W1, b1, W2, b2, W3, b3, W4, b4):
    lane = jnp.arange(_CW, dtype=jnp.int32)
    npad = _NCHP - _NCHT
    src = edge_index[0]
    dst = edge_index[1]
    phi_src = src * 2 - jnp.where(src >= _N // 2, _N - 1, 0)
    phi_dst = dst * 2 - jnp.where(dst >= _N // 2, _N - 1, 0)
    pad_src = jnp.broadcast_to(lane * 16, (npad, _CW)).reshape(-1)
    pad_dst = jnp.broadcast_to(_N + (lane & 7), (npad, _CW)).reshape(-1)
    src2 = jnp.concatenate([phi_src, pad_src]).reshape(_NCHP, _CW)
    dst2 = jnp.concatenate([phi_dst, pad_dst]).reshape(_NCHP, _CW)
    zeros_n = jnp.zeros((_N, _H), jnp.float32)

    b1x = jnp.concatenate([b1, b1]).reshape(1, 2 * _H)
    b2x = jnp.concatenate([b2, b2]).reshape(1, 2 * _H)
    b3x = jnp.concatenate([b3, b3]).reshape(1, 2 * _H)

    y2 = _mm_a(x, W1)
    p = _sc_aggregate(y2.reshape(_N, _H), zeros_n, src2, dst2)
    z2 = _mlp_b(p.reshape(_NC, _N // 2, 2 * _H), b1x, W2, b2x, W3)
    q = _sc_aggregate(z2.reshape(_N, _H), zeros_n, src2, dst2)
    return _mlp_c(q.reshape(_NC, _N // 2, 2 * _H), b3x, W4, b4.reshape(1, _D))


# speedup vs baseline: 1.0171x; 1.0171x over previous
import functools

import jax
import jax.numpy as jnp
from jax import lax
from jax.experimental import pallas as pl
from jax.experimental.pallas import tpu as pltpu
from jax.experimental.pallas import tpu_sc as plsc

_N = 10000
_D = 128
_H = 64
_E = 320000

_NC = 2
_NS = 16
_NW = _NC * _NS
_CW = 512
_NCHT = _E // _CW
_CPW = 20
_NCHP = _NW * _CPW
_STG = 24
_NDUMP = 8
_NIT = 10
_RPT = _N // _NIT


def _sc_aggregate(y, zeros_n, src2, dst2):
    mesh = plsc.VectorSubcoreMesh(
        core_axis_name="c", subcore_axis_name="s", num_cores=_NC, num_subcores=_NS
    )
    nacc = _N + _NDUMP

    @functools.partial(
        pl.kernel,
        mesh=mesh,
        out_type=jax.ShapeDtypeStruct((_NC, _N, _H), jnp.float32),
        scratch_types=[
            pltpu.VMEM((_STG, _CW), jnp.int32),
            pltpu.VMEM((_STG, _CW), jnp.int32),
            pltpu.VMEM((2, _CW, _H), jnp.float32),
            pltpu.VMEM_SHARED((nacc, _H), jnp.float32),
            pltpu.SemaphoreType.DMA,
            pltpu.SemaphoreType.DMA,
            pltpu.SemaphoreType.DMA,
            pltpu.SemaphoreType.DMA,
        ],
        compiler_params=pltpu.CompilerParams(use_tc_tiling_on_sc=False),
    )
    def agg(y_hbm, zero_hbm, src_hbm, dst_hbm, out_hbm, src_v, dst_v, rows_v, acc, g0, g1, s0, s1):
        c = lax.axis_index("c")
        s = lax.axis_index("s")
        w = s * _NC + c
        r0 = s * _RPT

        base = w * _CPW
        a0 = base - lax.rem(base, 8)
        off = base - a0
        pltpu.sync_copy(src_hbm.at[pl.ds(a0, _STG)], src_v)
        pltpu.sync_copy(dst_hbm.at[pl.ds(a0, _STG)], dst_v)

        @pl.when(jnp.logical_and(c == 0, s < _NIT))
        def _():
            pltpu.sync_copy(y_hbm.at[pl.ds(r0, _RPT)], acc.at[pl.ds(r0, _RPT)])

        @pl.when(jnp.logical_and(c == 1, s < _NIT))
        def _():
            pltpu.sync_copy(zero_hbm.at[pl.ds(r0, _RPT)], acc.at[pl.ds(r0, _RPT)])

        plsc.subcore_barrier()

        def g_start(j, slot, sem):
            return pltpu.async_copy(y_hbm.at[src_v.at[off + j]], rows_v.at[slot], sem)

        def g_wait(j, slot, sem):
            pltpu.make_async_copy(y_hbm.at[src_v.at[off + j]], rows_v.at[slot], sem).wait()

        def s_start(j, slot, sem):
            return pltpu.async_copy(rows_v.at[slot], acc.at[dst_v.at[off + j]], sem, add=True)

        def s_wait(j, slot, sem):
            pltpu.make_async_copy(rows_v.at[slot], acc.at[dst_v.at[off + j]], sem).wait()

        g_start(0, 0, g0)

        def body(g, carry):
            j0 = 2 * g
            j1 = j0 + 1

            @pl.when(g > 0)
            def _():
                s_wait(j1 - 2, 1, s1)

            g_start(j1, 1, g1)
            g_wait(j0, 0, g0)
            s_start(j0, 0, s0)
            s_wait(j0, 0, s0)

            @pl.when(g < _CPW // 2 - 1)
            def _():
                g_start(j0 + 2, 0, g0)

            g_wait(j1, 1, g1)
            s_start(j1, 1, s1)
            return carry

        lax.fori_loop(0, _CPW // 2, body, 0)
        s_wait(_CPW - 1, 1, s1)

        plsc.subcore_barrier()

        @pl.when(s < _NIT)
        def _():
            pltpu.sync_copy(acc.at[pl.ds(r0, _RPT)], out_hbm.at[c, pl.ds(r0, _RPT)])

    return agg(y, zeros_n, src2, dst2)


def _hdot(h, w):
    a = jnp.dot(h[:, : _H], w, preferred_element_type=jnp.float32)
    b = jnp.dot(h[:, _H :], w, preferred_element_type=jnp.float32)
    return jnp.concatenate([a, b], axis=1)


def _mm_a(x, w1):
    def body(x_ref, w_ref, o_ref):
        ya = jnp.dot(x_ref[: _N // 2], w_ref[...], preferred_element_type=jnp.float32)
        yb = jnp.dot(x_ref[_N // 2 :], w_ref[...], preferred_element_type=jnp.float32)
        o_ref[...] = jnp.concatenate([ya, yb], axis=1)

    return pl.pallas_call(
        body,
        out_shape=jax.ShapeDtypeStruct((_N // 2, 2 * _H), jnp.float32),
    )(x, w1)


def _mlp_b(p2, b1x, w2, b2x, w3):
    def body(p_ref, b1_ref, w2_ref, b2_ref, w3_ref, o_ref):
        h = jnp.maximum(p_ref[0] + p_ref[1] + b1_ref[...], 0.0)
        h = jnp.maximum(_hdot(h, w2_ref[...]) + b2_ref[...], 0.0)
        o_ref[...] = _hdot(h, w3_ref[...])

    return pl.pallas_call(
        body,
        out_shape=jax.ShapeDtypeStruct((_N // 2, 2 * _H), jnp.float32),
    )(p2, b1x, w2, b2x, w3)


def _mlp_c(q2, b3x, w4, b4r):
    def body(q_ref, b3_ref, w4_ref, b4_ref, o_ref):
        h = jnp.maximum(q_ref[0] + q_ref[1] + b3_ref[...], 0.0)
        o_ref[: _N // 2] = (
            jnp.dot(h[:, : _H], w4_ref[...], preferred_element_type=jnp.float32)
            + b4_ref[...]
        )
        o_ref[_N // 2 :] = (
            jnp.dot(h[:, _H :], w4_ref[...], preferred_element_type=jnp.float32)
            + b4_ref[...]
        )

    return pl.pallas_call(
        body,
        out_shape=jax.ShapeDtypeStruct((_N, _D), jnp.float32),
    )(q2, b3x, w4, b4r)


def kernel(x, edge_index, W1, b1, W2, b2, W3, b3, W4, b4):
    lane = jnp.arange(_CW, dtype=jnp.int32)
    npad = _NCHP - _NCHT
    src = edge_index[0]
    dst = edge_index[1]
    phi_src = src * 2 - jnp.where(src >= _N // 2, _N - 1, 0)
    phi_dst = dst * 2 - jnp.where(dst >= _N // 2, _N - 1, 0)
    pad_src = jnp.broadcast_to(lane * 16, (npad, _CW)).reshape(-1)
    pad_dst = jnp.broadcast_to(_N + (lane & 7), (npad, _CW)).reshape(-1)
    src2 = jnp.concatenate([phi_src, pad_src]).reshape(_NCHP, _CW)
    dst2 = jnp.concatenate([phi_dst, pad_dst]).reshape(_NCHP, _CW)
    zeros_n = jnp.zeros((_N, _H), jnp.float32)

    b1x = jnp.concatenate([b1, b1]).reshape(1, 2 * _H)
    b2x = jnp.concatenate([b2, b2]).reshape(1, 2 * _H)
    b3x = jnp.concatenate([b3, b3]).reshape(1, 2 * _H)

    y2 = _mm_a(x, W1)
    p = _sc_aggregate(y2.reshape(_N, _H), zeros_n, src2, dst2)
    z2 = _mlp_b(p.reshape(_NC, _N // 2, 2 * _H), b1x, W2, b2x, W3)
    q = _sc_aggregate(z2.reshape(_N, _H), zeros_n, src2, dst2)
    return _mlp_c(q.reshape(_NC, _N // 2, 2 * _H), b3x, W4, b4.reshape(1, _D))


# speedup vs baseline: 1.0264x; 1.0092x over previous
import functools

import jax
import jax.numpy as jnp
from jax import lax
from jax.experimental import pallas as pl
from jax.experimental.pallas import tpu as pltpu
from jax.experimental.pallas import tpu_sc as plsc

_N = 10000
_D = 128
_H = 64
_E = 320000

_NC = 2
_NS = 16
_NW = _NC * _NS
_CW = 512
_NCHT = _E // _CW
_CPW = 20
_NCHP = _NW * _CPW
_STG = 24
_NDUMP = 8
_NIT = 10
_RPT = _N // _NIT


def _sc_aggregate(y, zeros_n, src2, dst2):
    mesh = plsc.VectorSubcoreMesh(
        core_axis_name="c", subcore_axis_name="s", num_cores=_NC, num_subcores=_NS
    )
    nacc = _N + _NDUMP

    @functools.partial(
        pl.kernel,
        mesh=mesh,
        out_type=jax.ShapeDtypeStruct((_NC, _N, _H), jnp.float32),
        scratch_types=[
            pltpu.VMEM((_STG, _CW), jnp.int32),
            pltpu.VMEM((_STG, _CW), jnp.int32),
            pltpu.VMEM((2, _CW, _H), jnp.float32),
            pltpu.VMEM_SHARED((nacc, _H), jnp.float32),
            pltpu.SemaphoreType.DMA,
            pltpu.SemaphoreType.DMA,
            pltpu.SemaphoreType.DMA,
            pltpu.SemaphoreType.DMA,
        ],
        compiler_params=pltpu.CompilerParams(use_tc_tiling_on_sc=False),
    )
    def agg(y_hbm, zero_hbm, src_hbm, dst_hbm, out_hbm, src_v, dst_v, rows_v, acc, g0, g1, s0, s1):
        c = lax.axis_index("c")
        s = lax.axis_index("s")
        w = s * _NC + c
        r0 = s * _RPT

        base = w * _CPW
        a0 = base - lax.rem(base, 8)
        off = base - a0
        pltpu.sync_copy(src_hbm.at[pl.ds(a0, _STG)], src_v)
        pltpu.sync_copy(dst_hbm.at[pl.ds(a0, _STG)], dst_v)

        @pl.when(jnp.logical_and(c == 0, s < _NIT))
        def _():
            pltpu.sync_copy(y_hbm.at[pl.ds(r0, _RPT)], acc.at[pl.ds(r0, _RPT)])

        @pl.when(jnp.logical_and(c == 1, s < _NIT))
        def _():
            pltpu.sync_copy(zero_hbm.at[pl.ds(r0, _RPT)], acc.at[pl.ds(r0, _RPT)])

        plsc.subcore_barrier()

        def g_start(j, slot, sem):
            return pltpu.async_copy(y_hbm.at[src_v.at[off + j]], rows_v.at[slot], sem)

        def g_wait(j, slot, sem):
            pltpu.make_async_copy(y_hbm.at[src_v.at[off + j]], rows_v.at[slot], sem).wait()

        def s_start(j, slot, sem):
            return pltpu.async_copy(rows_v.at[slot], acc.at[dst_v.at[off + j]], sem, add=True)

        def s_wait(j, slot, sem):
            pltpu.make_async_copy(rows_v.at[slot], acc.at[dst_v.at[off + j]], sem).wait()

        g_start(0, 0, g0)

        def body(g, carry):
            j0 = 2 * g
            j1 = j0 + 1

            @pl.when(g > 0)
            def _():
                s_wait(j1 - 2, 1, s1)

            g_start(j1, 1, g1)
            g_wait(j0, 0, g0)
            s_start(j0, 0, s0)
            s_wait(j0, 0, s0)

            @pl.when(g < _CPW // 2 - 1)
            def _():
                g_start(j0 + 2, 0, g0)

            g_wait(j1, 1, g1)
            s_start(j1, 1, s1)
            return carry

        lax.fori_loop(0, _CPW // 2, body, 0)
        s_wait(_CPW - 1, 1, s1)

        plsc.subcore_barrier()

        @pl.when(s < _NIT)
        def _():
            pltpu.sync_copy(acc.at[pl.ds(r0, _RPT)], out_hbm.at[c, pl.ds(r0, _RPT)])

    return agg(y, zeros_n, src2, dst2)


def _bdiag(w):
    a, b = w.shape
    z = jnp.zeros((a, b), w.dtype)
    return jnp.concatenate(
        [jnp.concatenate([w, z], axis=1), jnp.concatenate([z, w], axis=1)], axis=0
    )


def _mm_a(x, w1):
    def body(x_ref, w_ref, o_ref):
        ya = jnp.dot(x_ref[: _N // 2], w_ref[...], preferred_element_type=jnp.float32)
        yb = jnp.dot(x_ref[_N // 2 :], w_ref[...], preferred_element_type=jnp.float32)
        o_ref[...] = jnp.concatenate([ya, yb], axis=1)

    return pl.pallas_call(
        body,
        out_shape=jax.ShapeDtypeStruct((_N // 2, 2 * _H), jnp.float32),
    )(x, w1)


def _mlp_b(p2, b1x, w2x, b2x, w3x):
    def body(p_ref, b1_ref, w2_ref, b2_ref, w3_ref, o_ref):
        h = jnp.maximum(p_ref[0] + p_ref[1] + b1_ref[...], 0.0)
        h = jnp.maximum(
            jnp.dot(h, w2_ref[...], preferred_element_type=jnp.float32) + b2_ref[...], 0.0
        )
        o_ref[...] = jnp.dot(h, w3_ref[...], preferred_element_type=jnp.float32)

    return pl.pallas_call(
        body,
        out_shape=jax.ShapeDtypeStruct((_N // 2, 2 * _H), jnp.float32),
    )(p2, b1x, w2x, b2x, w3x)


def _mlp_c(q2, b3x, w4, b4r):
    def body(q_ref, b3_ref, w4_ref, b4_ref, o_ref):
        h = jnp.maximum(q_ref[0] + q_ref[1] + b3_ref[...], 0.0)
        o_ref[: _N // 2] = (
            jnp.dot(h[:, : _H], w4_ref[...], preferred_element_type=jnp.float32)
            + b4_ref[...]
        )
        o_ref[_N // 2 :] = (
            jnp.dot(h[:, _H :], w4_ref[...], preferred_element_type=jnp.float32)
            + b4_ref[...]
        )

    return pl.pallas_call(
        body,
        out_shape=jax.ShapeDtypeStruct((_N, _D), jnp.float32),
    )(q2, b3x, w4, b4r)


def kernel(x, edge_index, W1, b1, W2, b2, W3, b3, W4, b4):
    lane = jnp.arange(_CW, dtype=jnp.int32)
    npad = _NCHP - _NCHT
    src = edge_index[0]
    dst = edge_index[1]
    phi_src = src * 2 - jnp.where(src >= _N // 2, _N - 1, 0)
    phi_dst = dst * 2 - jnp.where(dst >= _N // 2, _N - 1, 0)
    pad_src = jnp.broadcast_to(lane * 16, (npad, _CW)).reshape(-1)
    pad_dst = jnp.broadcast_to(_N + (lane & 7), (npad, _CW)).reshape(-1)
    src2 = jnp.concatenate([phi_src, pad_src]).reshape(_NCHP, _CW)
    dst2 = jnp.concatenate([phi_dst, pad_dst]).reshape(_NCHP, _CW)
    zeros_n = jnp.zeros((_N, _H), jnp.float32)

    b1x = jnp.concatenate([b1, b1]).reshape(1, 2 * _H)
    b2x = jnp.concatenate([b2, b2]).reshape(1, 2 * _H)
    b3x = jnp.concatenate([b3, b3]).reshape(1, 2 * _H)
    w2x = _bdiag(W2)
    w3x = _bdiag(W3)

    y2 = _mm_a(x, W1)
    p = _sc_aggregate(y2.reshape(_N, _H), zeros_n, src2, dst2)
    z2 = _mlp_b(p.reshape(_NC, _N // 2, 2 * _H), b1x, w2x, b2x, w3x)
    q = _sc_aggregate(z2.reshape(_N, _H), zeros_n, src2, dst2)
    return _mlp_c(q.reshape(_NC, _N // 2, 2 * _H), b3x, W4, b4.reshape(1, _D))


# speedup vs baseline: 1.0812x; 1.0533x over previous
import functools

import jax
import jax.numpy as jnp
from jax import lax
from jax.experimental import pallas as pl
from jax.experimental.pallas import tpu as pltpu
from jax.experimental.pallas import tpu_sc as plsc

_N = 10000
_D = 128
_H = 64
_E = 320000

_NC = 2
_NS = 16
_NW = _NC * _NS
_CW = 512
_NCHT = _E // _CW
_CPW = 20
_NCHP = _NW * _CPW
_STG = 24
_A31 = 616
_NDUMP = 8
_NIT = 10
_RPT = _N // _NIT


def _sc_aggregate(y, zeros_n, e2, pad2):
    mesh = plsc.VectorSubcoreMesh(
        core_axis_name="c", subcore_axis_name="s", num_cores=_NC, num_subcores=_NS
    )
    nacc = _N + _NDUMP

    @functools.partial(
        pl.kernel,
        mesh=mesh,
        out_type=jax.ShapeDtypeStruct((_NC, _N, _H), jnp.float32),
        scratch_types=[
            pltpu.VMEM((_STG, _CW), jnp.int32),
            pltpu.VMEM((_STG, _CW), jnp.int32),
            pltpu.VMEM((2, _CW, _H), jnp.float32),
            pltpu.VMEM_SHARED((nacc, _H), jnp.float32),
            pltpu.SemaphoreType.DMA,
            pltpu.SemaphoreType.DMA,
            pltpu.SemaphoreType.DMA,
            pltpu.SemaphoreType.DMA,
        ],
        compiler_params=pltpu.CompilerParams(use_tc_tiling_on_sc=False),
    )
    def agg(y_hbm, zero_hbm, e_hbm, pad_hbm, out_hbm, src_v, dst_v, rows_v, acc, g0, g1, s0, s1):
        c = lax.axis_index("c")
        s = lax.axis_index("s")
        w = s * _NC + c
        r0 = s * _RPT

        base = w * _CPW
        a0 = base - lax.rem(base, 8)
        off = base - a0

        @pl.when(w < _NW - 1)
        def _():
            pltpu.sync_copy(e_hbm.at[0, pl.ds(a0, _STG)], src_v)
            pltpu.sync_copy(e_hbm.at[1, pl.ds(a0, _STG)], dst_v)

        @pl.when(w == _NW - 1)
        def _():
            nreal = _NCHT - _A31
            npadc = _NCHP - _NCHT
            pltpu.sync_copy(e_hbm.at[0, pl.ds(_A31, nreal)], src_v.at[pl.ds(0, nreal)])
            pltpu.sync_copy(e_hbm.at[1, pl.ds(_A31, nreal)], dst_v.at[pl.ds(0, nreal)])
            pltpu.sync_copy(pad_hbm.at[0], src_v.at[pl.ds(nreal, npadc)])
            pltpu.sync_copy(pad_hbm.at[1], dst_v.at[pl.ds(nreal, npadc)])

        @pl.when(jnp.logical_and(c == 0, s < _NIT))
        def _():
            pltpu.sync_copy(y_hbm.at[pl.ds(r0, _RPT)], acc.at[pl.ds(r0, _RPT)])

        @pl.when(jnp.logical_and(c == 1, s < _NIT))
        def _():
            pltpu.sync_copy(zero_hbm.at[pl.ds(r0, _RPT)], acc.at[pl.ds(r0, _RPT)])

        plsc.subcore_barrier()

        def g_start(j, slot, sem):
            return pltpu.async_copy(y_hbm.at[src_v.at[off + j]], rows_v.at[slot], sem)

        def g_wait(j, slot, sem):
            pltpu.make_async_copy(y_hbm.at[src_v.at[off + j]], rows_v.at[slot], sem).wait()

        def s_start(j, slot, sem):
            return pltpu.async_copy(rows_v.at[slot], acc.at[dst_v.at[off + j]], sem, add=True)

        def s_wait(j, slot, sem):
            pltpu.make_async_copy(rows_v.at[slot], acc.at[dst_v.at[off + j]], sem).wait()

        g_start(0, 0, g0)

        def body(g, carry):
            j0 = 2 * g
            j1 = j0 + 1

            @pl.when(g > 0)
            def _():
                s_wait(j1 - 2, 1, s1)

            g_start(j1, 1, g1)
            g_wait(j0, 0, g0)
            s_start(j0, 0, s0)
            s_wait(j0, 0, s0)

            @pl.when(g < _CPW // 2 - 1)
            def _():
                g_start(j0 + 2, 0, g0)

            g_wait(j1, 1, g1)
            s_start(j1, 1, s1)
            return carry

        lax.fori_loop(0, _CPW // 2, body, 0)
        s_wait(_CPW - 1, 1, s1)

        plsc.subcore_barrier()

        @pl.when(s < _NIT)
        def _():
            pltpu.sync_copy(acc.at[pl.ds(r0, _RPT)], out_hbm.at[c, pl.ds(r0, _RPT)])

    return agg(y, zeros_n, e2, pad2)


def _edge_repack(edge_index):
    def body(e_ref, o_ref):
        e = e_ref[...]
        phi = e * 2 - jnp.where(e >= _N // 2, _N - 1, 0).astype(jnp.int32)
        o_ref[...] = phi.reshape(2, _NCHT, _CW)

    return pl.pallas_call(
        body,
        out_shape=jax.ShapeDtypeStruct((2, _NCHT, _CW), jnp.int32),
    )(edge_index)


def _bdiag(w):
    a, b = w.shape
    z = jnp.zeros((a, b), w.dtype)
    return jnp.concatenate(
        [jnp.concatenate([w, z], axis=1), jnp.concatenate([z, w], axis=1)], axis=0
    )


def _mm_a(x, w1):
    def body(x_ref, w_ref, o_ref):
        ya = jnp.dot(x_ref[: _N // 2], w_ref[...], preferred_element_type=jnp.float32)
        yb = jnp.dot(x_ref[_N // 2 :], w_ref[...], preferred_element_type=jnp.float32)
        o_ref[...] = jnp.concatenate([ya, yb], axis=1)

    return pl.pallas_call(
        body,
        out_shape=jax.ShapeDtypeStruct((_N // 2, 2 * _H), jnp.float32),
    )(x, w1)


def _mlp_b(p2, b1x, w2x, b2x, w3x):
    def body(p_ref, b1_ref, w2_ref, b2_ref, w3_ref, o_ref):
        h = jnp.maximum(p_ref[0] + p_ref[1] + b1_ref[...], 0.0)
        h = jnp.maximum(
            jnp.dot(h, w2_ref[...], preferred_element_type=jnp.float32) + b2_ref[...], 0.0
        )
        o_ref[...] = jnp.dot(h, w3_ref[...], preferred_element_type=jnp.float32)

    return pl.pallas_call(
        body,
        out_shape=jax.ShapeDtypeStruct((_N // 2, 2 * _H), jnp.float32),
    )(p2, b1x, w2x, b2x, w3x)


def _mlp_c(q2, b3x, w4, b4r):
    def body(q_ref, b3_ref, w4_ref, b4_ref, o_ref):
        h = jnp.maximum(q_ref[0] + q_ref[1] + b3_ref[...], 0.0)
        o_ref[: _N // 2] = (
            jnp.dot(h[:, : _H], w4_ref[...], preferred_element_type=jnp.float32)
            + b4_ref[...]
        )
        o_ref[_N // 2 :] = (
            jnp.dot(h[:, _H :], w4_ref[...], preferred_element_type=jnp.float32)
            + b4_ref[...]
        )

    return pl.pallas_call(
        body,
        out_shape=jax.ShapeDtypeStruct((_N, _D), jnp.float32),
    )(q2, b3x, w4, b4r)


def kernel(x, edge_index, W1, b1, W2, b2, W3, b3, W4, b4):
    lane = jnp.arange(_CW, dtype=jnp.int32)
    npad = _NCHP - _NCHT
    e2 = _edge_repack(edge_index)
    pad2 = jnp.stack(
        [
            jnp.broadcast_to(lane * 16, (npad, _CW)),
            jnp.broadcast_to(_N + (lane & 7), (npad, _CW)),
        ]
    )
    zeros_n = jnp.zeros((_N, _H), jnp.float32)

    b1x = jnp.concatenate([b1, b1]).reshape(1, 2 * _H)
    b2x = jnp.concatenate([b2, b2]).reshape(1, 2 * _H)
    b3x = jnp.concatenate([b3, b3]).reshape(1, 2 * _H)
    w2x = _bdiag(W2)
    w3x = _bdiag(W3)

    y2 = _mm_a(x, W1)
    p = _sc_aggregate(y2.reshape(_N, _H), zeros_n, e2, pad2)
    z2 = _mlp_b(p.reshape(_NC, _N // 2, 2 * _H), b1x, w2x, b2x, w3x)
    q = _sc_aggregate(z2.reshape(_N, _H), zeros_n, e2, pad2)
    return _mlp_c(q.reshape(_NC, _N // 2, 2 * _H), b3x, W4, b4.reshape(1, _D))


# speedup vs baseline: 1.0849x; 1.0034x over previous
import functools

import jax
import jax.numpy as jnp
from jax import lax
from jax.experimental import pallas as pl
from jax.experimental.pallas import tpu as pltpu
from jax.experimental.pallas import tpu_sc as plsc

_N = 10000
_D = 128
_H = 64
_E = 320000

_NC = 2
_NS = 16
_NW = _NC * _NS
_CW = 512
_NCHT = _E // _CW
_CPW = 20
_NCHP = _NW * _CPW
_STG = 24
_A31 = 616
_NDUMP = 8
_NIT = 10
_RPT = _N // _NIT


def _sc_aggregate(y, e2, pad2):
    mesh = plsc.VectorSubcoreMesh(
        core_axis_name="c", subcore_axis_name="s", num_cores=_NC, num_subcores=_NS
    )
    nacc = _N + _NDUMP

    @functools.partial(
        pl.kernel,
        mesh=mesh,
        out_type=jax.ShapeDtypeStruct((_NC, _N, _H), jnp.float32),
        scratch_types=[
            pltpu.VMEM((_STG, _CW), jnp.int32),
            pltpu.VMEM((_STG, _CW), jnp.int32),
            pltpu.VMEM((2, _CW, _H), jnp.float32),
            pltpu.VMEM_SHARED((nacc, _H), jnp.float32),
            pltpu.SemaphoreType.DMA,
            pltpu.SemaphoreType.DMA,
            pltpu.SemaphoreType.DMA,
            pltpu.SemaphoreType.DMA,
        ],
        compiler_params=pltpu.CompilerParams(use_tc_tiling_on_sc=False),
    )
    def agg(y_hbm, e_hbm, pad_hbm, out_hbm, src_v, dst_v, rows_v, acc, g0, g1, s0, s1):
        c = lax.axis_index("c")
        s = lax.axis_index("s")
        w = s * _NC + c
        r0 = s * _RPT

        base = w * _CPW
        a0 = base - lax.rem(base, 8)
        off = base - a0

        @pl.when(w < _NW - 1)
        def _():
            pltpu.sync_copy(e_hbm.at[0, pl.ds(a0, _STG)], src_v)
            pltpu.sync_copy(e_hbm.at[1, pl.ds(a0, _STG)], dst_v)

        @pl.when(w == _NW - 1)
        def _():
            nreal = _NCHT - _A31
            npadc = _NCHP - _NCHT
            pltpu.sync_copy(e_hbm.at[0, pl.ds(_A31, nreal)], src_v.at[pl.ds(0, nreal)])
            pltpu.sync_copy(e_hbm.at[1, pl.ds(_A31, nreal)], dst_v.at[pl.ds(0, nreal)])
            pltpu.sync_copy(pad_hbm.at[0], src_v.at[pl.ds(nreal, npadc)])
            pltpu.sync_copy(pad_hbm.at[1], dst_v.at[pl.ds(nreal, npadc)])

        @pl.when(s < _NIT)
        def _():
            pltpu.sync_copy(y_hbm.at[pl.ds(r0, _RPT)], acc.at[pl.ds(r0, _RPT)])

        plsc.subcore_barrier()

        def g_start(j, slot, sem):
            return pltpu.async_copy(y_hbm.at[src_v.at[off + j]], rows_v.at[slot], sem)

        def g_wait(j, slot, sem):
            pltpu.make_async_copy(y_hbm.at[src_v.at[off + j]], rows_v.at[slot], sem).wait()

        def s_start(j, slot, sem):
            return pltpu.async_copy(rows_v.at[slot], acc.at[dst_v.at[off + j]], sem, add=True)

        def s_wait(j, slot, sem):
            pltpu.make_async_copy(rows_v.at[slot], acc.at[dst_v.at[off + j]], sem).wait()

        g_start(0, 0, g0)

        def body(g, carry):
            j0 = 2 * g
            j1 = j0 + 1

            @pl.when(g > 0)
            def _():
                s_wait(j1 - 2, 1, s1)

            g_start(j1, 1, g1)
            g_wait(j0, 0, g0)
            s_start(j0, 0, s0)
            s_wait(j0, 0, s0)

            @pl.when(g < _CPW // 2 - 1)
            def _():
                g_start(j0 + 2, 0, g0)

            g_wait(j1, 1, g1)
            s_start(j1, 1, s1)
            return carry

        lax.fori_loop(0, _CPW // 2, body, 0)
        s_wait(_CPW - 1, 1, s1)

        plsc.subcore_barrier()

        @pl.when(s < _NIT)
        def _():
            pltpu.sync_copy(acc.at[pl.ds(r0, _RPT)], out_hbm.at[c, pl.ds(r0, _RPT)])

    return agg(y, e2, pad2)


def _edge_repack(edge_index):
    def body(e_ref, o_ref):
        e = e_ref[...]
        phi = e * 2 - jnp.where(e >= _N // 2, _N - 1, 0).astype(jnp.int32)
        o_ref[...] = phi.reshape(2, _NCHT, _CW)

    return pl.pallas_call(
        body,
        out_shape=jax.ShapeDtypeStruct((2, _NCHT, _CW), jnp.int32),
    )(edge_index)


def _bdiag(w):
    a, b = w.shape
    z = jnp.zeros((a, b), w.dtype)
    return jnp.concatenate(
        [jnp.concatenate([w, z], axis=1), jnp.concatenate([z, w], axis=1)], axis=0
    )


def _mm_a(x, w1):
    def body(x_ref, w_ref, o_ref):
        ya = jnp.dot(x_ref[: _N // 2], w_ref[...], preferred_element_type=jnp.float32)
        yb = jnp.dot(x_ref[_N // 2 :], w_ref[...], preferred_element_type=jnp.float32)
        o_ref[...] = jnp.concatenate([ya, yb], axis=1)

    return pl.pallas_call(
        body,
        out_shape=jax.ShapeDtypeStruct((_N // 2, 2 * _H), jnp.float32),
    )(x, w1)


def _mlp_b(p2, y2, b1x, w2x, b2x, w3x):
    def body(p_ref, y_ref, b1_ref, w2_ref, b2_ref, w3_ref, o_ref):
        h = jnp.maximum(p_ref[0] + p_ref[1] - y_ref[...] + b1_ref[...], 0.0)
        h = jnp.maximum(
            jnp.dot(h, w2_ref[...], preferred_element_type=jnp.float32) + b2_ref[...], 0.0
        )
        o_ref[...] = jnp.dot(h, w3_ref[...], preferred_element_type=jnp.float32)

    return pl.pallas_call(
        body,
        out_shape=jax.ShapeDtypeStruct((_N // 2, 2 * _H), jnp.float32),
    )(p2, y2, b1x, w2x, b2x, w3x)


def _mlp_c(q2, z2, b3x, w4, b4r):
    def body(q_ref, z_ref, b3_ref, w4_ref, b4_ref, o_ref):
        h = jnp.maximum(q_ref[0] + q_ref[1] - z_ref[...] + b3_ref[...], 0.0)
        o_ref[: _N // 2] = (
            jnp.dot(h[:, : _H], w4_ref[...], preferred_element_type=jnp.float32)
            + b4_ref[...]
        )
        o_ref[_N // 2 :] = (
            jnp.dot(h[:, _H :], w4_ref[...], preferred_element_type=jnp.float32)
            + b4_ref[...]
        )

    return pl.pallas_call(
        body,
        out_shape=jax.ShapeDtypeStruct((_N, _D), jnp.float32),
    )(q2, z2, b3x, w4, b4r)


def kernel(x, edge_index, W1, b1, W2, b2, W3, b3, W4, b4):
    lane = jnp.arange(_CW, dtype=jnp.int32)
    npad = _NCHP - _NCHT
    e2 = _edge_repack(edge_index)
    pad2 = jnp.stack(
        [
            jnp.broadcast_to(lane * 16, (npad, _CW)),
            jnp.broadcast_to(_N + (lane & 7), (npad, _CW)),
        ]
    )

    b1x = jnp.concatenate([b1, b1]).reshape(1, 2 * _H)
    b2x = jnp.concatenate([b2, b2]).reshape(1, 2 * _H)
    b3x = jnp.concatenate([b3, b3]).reshape(1, 2 * _H)
    w2x = _bdiag(W2)
    w3x = _bdiag(W3)

    y2 = _mm_a(x, W1)
    p = _sc_aggregate(y2.reshape(_N, _H), e2, pad2)
    z2 = _mlp_b(p.reshape(_NC, _N // 2, 2 * _H), y2, b1x, w2x, b2x, w3x)
    q = _sc_aggregate(z2.reshape(_N, _H), e2, pad2)
    return _mlp_c(q.reshape(_NC, _N // 2, 2 * _H), z2, b3x, W4, b4.reshape(1, _D))
